# baseline (speedup 1.0000x reference)
import functools

import jax
import jax.numpy as jnp
from jax import lax
from jax.experimental import pallas as pl
from jax.experimental.pallas import tpu as pltpu
from jax.experimental.pallas import tpu_sc as plsc

N0, N1, N2 = 50000, 10000, 4096
E1, E2 = 320000, 131072
D = 128
CW = 16
NC, NS = 2, 16


def _make_sc_agg(E, NP, C, S):
    EW = E // (NC * NS)
    n_chunks = EW // C
    assert n_chunks * C == EW
    n_rounds = n_chunks // S
    assert n_rounds * S == n_chunks and n_rounds >= 2
    RPT = NP // NS
    assert RPT * NS == NP and RPT % 8 == 0
    mesh = plsc.VectorSubcoreMesh(core_axis_name="c", subcore_axis_name="s",
                                  num_cores=NC, num_subcores=NS)

    @functools.partial(
        pl.kernel,
        out_type=jax.ShapeDtypeStruct((NC, NP, D), jnp.float32),
        mesh=mesh,
        scratch_types=(
            [pltpu.VMEM((S, C, D), jnp.float32),
             pltpu.VMEM((EW,), jnp.int32),
             pltpu.VMEM((EW,), jnp.int32),
             pltpu.VMEM_SHARED((NP, D), jnp.float32)]
            + [pltpu.SemaphoreType.DMA] * (2 * S)
        ),
    )
    def agg_kernel(table, src, dst, zeros, out, bufs, idxs_v, idxd_v,
                   acc_sh, *sems):
        gsem = sems[:S]
        ssem = sems[S:]
        cid = lax.axis_index("c")
        sid = lax.axis_index("s")
        base = (cid * NS + sid) * EW
        pltpu.sync_copy(src.at[pl.ds(base, EW)], idxs_v)
        pltpu.sync_copy(dst.at[pl.ds(base, EW)], idxd_v)
        pltpu.sync_copy(zeros.at[pl.ds(sid * RPT, RPT)],
                        acc_sh.at[pl.ds(sid * RPT, RPT)])
        plsc.subcore_barrier()

        def gather(c, s):
            return pltpu.async_copy(table.at[idxs_v.at[pl.ds(c * C, C)]],
                                    bufs.at[s], gsem[s])

        def scatter(c, s):
            return pltpu.async_copy(bufs.at[s],
                                    acc_sh.at[idxd_v.at[pl.ds(c * C, C)]],
                                    ssem[s], add=True)

        for s in range(S):
            gather(s, s)

        def round_body(r, carry):
            for s in range(S):
                c = r * S + s
                pltpu.make_async_copy(table.at[idxs_v.at[pl.ds(c * C, C)]],
                                      bufs.at[s], gsem[s]).wait()
                scatter(c, s)
                pltpu.make_async_copy(bufs.at[s],
                                      acc_sh.at[idxd_v.at[pl.ds(c * C, C)]],
                                      ssem[s]).wait()

                @pl.when(r < n_rounds - 1)
                def _():
                    gather(c + S, s)
            return carry

        lax.fori_loop(0, n_rounds, round_body, 0)
        plsc.subcore_barrier()
        pltpu.sync_copy(acc_sh.at[pl.ds(sid * RPT, RPT)],
                        out.at[cid, pl.ds(sid * RPT, RPT)])

    return agg_kernel


def _make_sc_cnt(E, NP, C, S):
    EW = E // (NC * NS)
    n_chunks = EW // C
    assert n_chunks * C == EW
    RPT = NP // NS
    assert RPT * NS == NP and RPT % 8 == 0
    mesh = plsc.VectorSubcoreMesh(core_axis_name="c", subcore_axis_name="s",
                                  num_cores=NC, num_subcores=NS)

    @functools.partial(
        pl.kernel,
        out_type=jax.ShapeDtypeStruct((NC, NP, CW), jnp.float32),
        mesh=mesh,
        scratch_types=[
            pltpu.VMEM((n_chunks, C), jnp.int32),
            pltpu.VMEM((C, CW), jnp.float32),
            pltpu.VMEM_SHARED((NP, CW), jnp.float32),
            pltpu.SemaphoreType.DMA,
        ],
        compiler_params=pltpu.CompilerParams(use_tc_tiling_on_sc=False),
    )
    def cnt_kernel(dstR, ones, zeros, out, idxd_v, ones_v, acc_sh, ssem):
        cid = lax.axis_index("c")
        sid = lax.axis_index("s")
        w = cid * NS + sid
        pltpu.sync_copy(zeros.at[pl.ds(sid * RPT, RPT)],
                        acc_sh.at[pl.ds(sid * RPT, RPT)])
        pltpu.sync_copy(dstR.at[pl.ds(w * n_chunks, n_chunks)], idxd_v)
        pltpu.sync_copy(ones, ones_v)
        plsc.subcore_barrier()

        def scatter(c):
            return pltpu.async_copy(ones_v, acc_sh.at[idxd_v.at[c]], ssem,
                                    add=True)

        for c in range(S):
            scatter(c)

        def body(c, carry):
            pltpu.make_async_copy(ones_v, acc_sh.at[idxd_v.at[c]],
                                  ssem).wait()
            scatter(c + S)
            return carry

        lax.fori_loop(0, n_chunks - S, body, 0)
        for c in range(S):
            pltpu.make_async_copy(ones_v, acc_sh.at[idxd_v.at[0]],
                                  ssem).wait()
        plsc.subcore_barrier()
        pltpu.sync_copy(acc_sh.at[pl.ds(sid * RPT, RPT)],
                        out.at[cid, pl.ds(sid * RPT, RPT)])

    return cnt_kernel


def _dense(parts, cnts, xdst, wlT, wrT, b, relu, BR, N):
    assert N % BR == 0

    def body(p_ref, c_ref, xd_ref, wl_ref, wr_ref, b_ref, o_ref):
        agg = p_ref[0] + p_ref[1]
        cnt = jnp.maximum(c_ref[0, :, 0:1] + c_ref[1, :, 0:1], 1.0)
        mean = agg / cnt
        h = jnp.dot(mean, wl_ref[...], preferred_element_type=jnp.float32)
        h = h + jnp.dot(xd_ref[...], wr_ref[...],
                        preferred_element_type=jnp.float32)
        h = h + b_ref[...]
        if relu:
            h = jnp.maximum(h, 0.0)
        o_ref[...] = h

    return pl.pallas_call(
        body,
        grid=(N // BR,),
        in_specs=[
            pl.BlockSpec((NC, BR, D), lambda i: (0, i, 0)),
            pl.BlockSpec((NC, BR, CW), lambda i: (0, i, 0)),
            pl.BlockSpec((BR, D), lambda i: (i, 0)),
            pl.BlockSpec((D, D), lambda i: (0, 0)),
            pl.BlockSpec((D, D), lambda i: (0, 0)),
            pl.BlockSpec((1, D), lambda i: (0, 0)),
        ],
        out_specs=pl.BlockSpec((BR, D), lambda i: (i, 0)),
        out_shape=jax.ShapeDtypeStruct((N, D), jnp.float32),
    )(parts, cnts, xdst, wlT, wrT, b)


def kernel(x, edge_index1, edge_index2, W_l1, b_l1, W_r1, W_l2, b_l2, W_r2):
    src1 = edge_index1[0].astype(jnp.int32)
    dst1 = edge_index1[1].astype(jnp.int32)
    src2 = edge_index2[0].astype(jnp.int32)
    dst2 = edge_index2[1].astype(jnp.int32)

    N1P = 10112
    z1 = jnp.zeros((N1P, D), jnp.float32)
    z2 = jnp.zeros((N2, D), jnp.float32)
    zc1 = jnp.zeros((N1P, CW), jnp.float32)
    zc2 = jnp.zeros((N2, CW), jnp.float32)
    ones1 = jnp.ones((40, CW), jnp.float32)
    ones2 = jnp.ones((64, CW), jnp.float32)

    parts1 = _make_sc_agg(E1, N1P, 40, 5)(x, src1, dst1, z1)
    cnts1 = _make_sc_cnt(E1, N1P, 40, 8)(dst1.reshape(-1, 40), ones1, zc1)
    he = _dense(parts1, cnts1, x, W_l1.T, W_r1.T, b_l1[None, :],
                relu=True, BR=2000, N=N1)
    parts2 = _make_sc_agg(E2, N2, 64, 8)(he, src2, dst2, z2)
    cnts2 = _make_sc_cnt(E2, N2, 64, 8)(dst2.reshape(-1, 64), ones2, zc2)
    h2 = _dense(parts2, cnts2, he, W_l2.T, W_r2.T, b_l2[None, :],
                relu=False, BR=2048, N=N2)
    return (h2, h2, he)
